# baseline (speedup 1.0000x reference)
import functools

import jax
import jax.numpy as jnp
from jax import lax
from jax.experimental import pallas as pl
from jax.experimental.pallas import tpu as pltpu
from jax.experimental.pallas import tpu_sc as plsc

NUM_CORES = 2
NUM_SUBCORES = 16
LANES = 16
NW = NUM_CORES * NUM_SUBCORES

BATCH = 16384
K = 64
D = 16
PACK = 128
B_PER_W = BATCH // NW
CH = 64
N_CHUNKS = B_PER_W // CH
GROUPS = CH // LANES
NCH = BATCH // CH
PF = 8
PBITS = 18

M_SU = 0
M_SI = 1
M_ORD = 2
M_NF = 3
M_RW = 4
M_PF = 5

IT_F = 0
IT_FP = D
IT_GI = 2 * D
OUT_TU = K
OUT_XUI = K + D


def _project_body(f_ref, w_ref, b_ref, out_ref):
    out_ref[...] = lax.dot_general(
        f_ref[...], w_ref[...],
        dimension_numbers=(((1,), (1,)), ((), ())),
        preferred_element_type=jnp.float32,
    ) + b_ref[...]


def _project(F, W, b):
    return pl.pallas_call(
        _project_body,
        out_shape=jax.ShapeDtypeStruct((F.shape[0], W.shape[0]), jnp.float32),
    )(F, W, b.reshape(1, -1))


def _sc_body(meta_hbm, pro_hbm, gut_hbm, tut_hbm, it_hbm,
             guo_hbm, ito_hbm,
             su_v, si_v, ord_v, nf_v, rw_v, pf_v, pro_v,
             gu_ar, tu_ar, it_v, guo_v,
             sem_it, sem_out, sem_sl):
    wid = lax.axis_index("s") * NUM_CORES + lax.axis_index("c")
    iot = lax.iota(jnp.int32, LANES)

    def slab_fetch(slab_id, slot):
        colb = pl.multiple_of(slab_id * PACK, PACK)
        gslot = pl.multiple_of(slot * K, K)
        tslot = pl.multiple_of(slot * D, D)
        pltpu.async_copy(gut_hbm.at[:, pl.ds(colb, PACK)],
                         gu_ar.at[pl.ds(gslot, K), :], sem_sl.at[slot])
        pltpu.async_copy(tut_hbm.at[:, pl.ds(colb, PACK)],
                         tu_ar.at[pl.ds(tslot, D), :], sem_sl.at[slot])

    def slab_drain(slot):
        gslot = pl.multiple_of(slot * K, K)
        tslot = pl.multiple_of(slot * D, D)
        pltpu.make_async_copy(gut_hbm.at[:, pl.ds(0, PACK)],
                              gu_ar.at[pl.ds(gslot, K), :],
                              sem_sl.at[slot]).wait()
        pltpu.make_async_copy(tut_hbm.at[:, pl.ds(0, PACK)],
                              tu_ar.at[pl.ds(tslot, D), :],
                              sem_sl.at[slot]).wait()

    pltpu.sync_copy(pro_hbm.at[wid], pro_v)
    provec = pro_v[0, pl.ds(0, LANES)]
    for i in range(PF - 1):
        slab_fetch(provec[i], i)

    def chunk_body(c, rw_last):
        ch = wid * N_CHUNKS + c
        pltpu.sync_copy(meta_hbm.at[M_SU, ch], su_v)
        pltpu.sync_copy(meta_hbm.at[M_SI, ch], si_v)
        pltpu.sync_copy(meta_hbm.at[M_ORD, ch], ord_v)
        pltpu.sync_copy(meta_hbm.at[M_NF, ch], nf_v)
        pltpu.sync_copy(meta_hbm.at[M_RW, ch], rw_v)
        pltpu.sync_copy(meta_hbm.at[M_PF, ch], pf_v)
        it_cp = pltpu.async_copy(it_hbm.at[si_v.at[0]], it_v, sem_it)

        for g in range(GROUPS):
            sl = pl.ds(g * LANES, LANES)
            su_vec = su_v[0, sl]
            nf_vec = nf_v[0, sl]
            rw_vec = rw_v[0, sl]
            pf_vec = pf_v[0, sl]
            for l in range(LANES):
                lg = g * LANES + l
                su_s = su_vec[l]
                w_slot = rw_vec[l]

                @pl.when(nf_vec[l] != 0)
                def _rotate():
                    slab_drain(w_slot)
                    slab_fetch(pf_vec[l], (w_slot + PF - 1) & (PF - 1))

                colv = jnp.full((LANES,), su_s & (PACK - 1), jnp.int32)
                gbase = w_slot * K
                for q in range(K // LANES):
                    guo_v[lg, pl.ds(q * LANES, LANES)] = plsc.load_gather(
                        gu_ar, [gbase + iot + q * LANES, colv])
                guo_v[lg, pl.ds(OUT_TU, D)] = plsc.load_gather(
                    tu_ar, [w_slot * D + iot, colv])

        it_cp.wait()

        for g in range(GROUPS):
            rows = iot + g * LANES
            acc = jnp.zeros((LANES,), jnp.float32)
            for k in range(K):
                acc = acc + (
                    plsc.load_gather(
                        guo_v, [rows, jnp.full((LANES,), k, jnp.int32)])
                    * plsc.load_gather(
                        it_v, [rows, jnp.full((LANES,), IT_GI + k, jnp.int32)]))
            for dd in range(D):
                acc = acc + (
                    plsc.load_gather(
                        guo_v, [rows, jnp.full((LANES,), OUT_TU + dd, jnp.int32)])
                    * plsc.load_gather(
                        it_v, [rows, jnp.full((LANES,), IT_FP + dd, jnp.int32)]))
            plsc.store_scatter(
                guo_v, [rows, jnp.full((LANES,), OUT_XUI, jnp.int32)], acc)

        pltpu.async_copy(guo_v, guo_hbm.at[ord_v.at[0]], sem_out).wait()
        pltpu.async_copy(it_v, ito_hbm.at[ord_v.at[0]], sem_out).wait()
        return rw_v[0, pl.ds(CH - LANES, LANES)][LANES - 1]

    rw_last = lax.fori_loop(0, N_CHUNKS, chunk_body, jnp.int32(0))

    for i in range(PF - 1):
        slab_drain((rw_last + 1 + i) & (PF - 1))


@functools.partial(
    pl.kernel,
    out_type=(
        jax.ShapeDtypeStruct((BATCH, PACK), jnp.float32),
        jax.ShapeDtypeStruct((BATCH, PACK), jnp.float32),
    ),
    mesh=plsc.VectorSubcoreMesh(core_axis_name="c", subcore_axis_name="s"),
    compiler_params=pltpu.CompilerParams(
        needs_layout_passes=False, use_tc_tiling_on_sc=True),
    scratch_types=[
        pltpu.VMEM((1, CH), jnp.int32),
        pltpu.VMEM((1, CH), jnp.int32),
        pltpu.VMEM((1, CH), jnp.int32),
        pltpu.VMEM((1, CH), jnp.int32),
        pltpu.VMEM((1, CH), jnp.int32),
        pltpu.VMEM((1, CH), jnp.int32),
        pltpu.VMEM((1, PACK), jnp.int32),
        pltpu.VMEM((PF * K, PACK), jnp.float32),
        pltpu.VMEM((PF * D, PACK), jnp.float32),
        pltpu.VMEM((CH, PACK), jnp.float32),
        pltpu.VMEM((CH, PACK), jnp.float32),
        pltpu.SemaphoreType.DMA,
        pltpu.SemaphoreType.DMA,
        pltpu.SemaphoreType.DMA((PF,)),
    ],
)
def _sc_kernel(*refs):
    _sc_body(*refs)


def kernel(users, items, Gu, Gi, Tu, F, W, b):
    u = users[:, 0]
    it = items[:, 0]
    fp = _project(F, W, b)
    itab = jnp.pad(jnp.concatenate([F, fp, Gi], axis=1),
                   ((0, 0), (0, PACK - 2 * D - K)))

    pos = lax.iota(jnp.int32, BATCH)
    skey = jnp.sort(
        jnp.bitwise_or(jnp.left_shift(lax.shift_right_logical(u, 7), PBITS),
                       pos))
    order = jnp.bitwise_and(skey, (1 << PBITS) - 1)
    slab = lax.shift_right_logical(skey, PBITS)
    su = jnp.take(u, order)
    si = jnp.take(it, order)
    nf = jnp.where((pos % B_PER_W == 0) | (slab != jnp.roll(slab, 1)),
                   1, 0).astype(jnp.int32)
    runid = jnp.cumsum(nf) - 1
    sor = jnp.zeros((BATCH,), jnp.int32).at[runid].set(slab)
    pfs = jnp.take(sor, jnp.clip(runid + PF - 1, 0, BATCH - 1))
    runid0 = jnp.take(runid, (pos // B_PER_W) * B_PER_W)
    rw = (runid - runid0) & (PF - 1)
    meta = jnp.stack([su, si, order, nf, rw, pfs]).reshape(6, NCH, 1, CH)
    pro = jnp.take(sor, jnp.clip(
        runid[::B_PER_W][:, None]
        + jnp.arange(PF - 1, dtype=jnp.int32)[None, :], 0, BATCH - 1))
    pro3 = jnp.zeros((NW, 1, PACK), jnp.int32).at[:, 0, :PF - 1].set(pro)

    guo, ito = _sc_kernel(meta, pro3, Gu.T, Tu.T, itab)
    xui = guo[:, OUT_XUI]
    gamma_u = guo[:, :K]
    gamma_i = ito[:, IT_GI:IT_GI + K]
    theta_u = guo[:, OUT_TU:OUT_TU + D]
    effe_i = ito[:, IT_F:IT_F + D]
    return (xui, gamma_u, gamma_i, theta_u, effe_i)


# speedup vs baseline: 1.0774x; 1.0774x over previous
import functools

import jax
import jax.numpy as jnp
from jax import lax
from jax.experimental import pallas as pl
from jax.experimental.pallas import tpu as pltpu
from jax.experimental.pallas import tpu_sc as plsc

NUM_CORES = 2
NUM_SUBCORES = 16
LANES = 16
NW = NUM_CORES * NUM_SUBCORES

BATCH = 16384
K = 64
D = 16
PACK = 128
B_PER_W = BATCH // NW
CH = 64
N_CHUNKS = B_PER_W // CH
GROUPS = CH // LANES
NCH = BATCH // CH
PF = 8
PBITS = 18

M_SU = 0
M_SI = 1
M_ORD = 2
M_NF = 3
M_RW = 4
M_PF = 5

IT_F = 0
IT_FP = D
IT_GI = 2 * D
OUT_TU = K
OUT_XUI = K + D


def _project_body(f_ref, w_ref, b_ref, out_ref):
    out_ref[...] = lax.dot_general(
        f_ref[...], w_ref[...],
        dimension_numbers=(((1,), (1,)), ((), ())),
        preferred_element_type=jnp.float32,
    ) + b_ref[...]


def _project(F, W, b):
    return pl.pallas_call(
        _project_body,
        out_shape=jax.ShapeDtypeStruct((F.shape[0], W.shape[0]), jnp.float32),
    )(F, W, b.reshape(1, -1))


def _sc_body(meta_hbm, pro_hbm, gut_hbm, tut_hbm, it_hbm,
             guo_hbm, ito_hbm,
             meta_v, pro_v,
             gu_ar, tu_ar, it_v, guo_v,
             sem_it, sem_out, sem_sl):
    wid = lax.axis_index("s") * NUM_CORES + lax.axis_index("c")
    iot = lax.iota(jnp.int32, LANES)

    def slab_fetch(slab_id, slot):
        colb = pl.multiple_of(slab_id * PACK, PACK)
        gslot = pl.multiple_of(slot * K, K)
        tslot = pl.multiple_of(slot * D, D)
        pltpu.async_copy(gut_hbm.at[:, pl.ds(colb, PACK)],
                         gu_ar.at[pl.ds(gslot, K), :], sem_sl.at[slot])
        pltpu.async_copy(tut_hbm.at[:, pl.ds(colb, PACK)],
                         tu_ar.at[pl.ds(tslot, D), :], sem_sl.at[slot])

    def slab_drain(slot):
        gslot = pl.multiple_of(slot * K, K)
        tslot = pl.multiple_of(slot * D, D)
        pltpu.make_async_copy(gut_hbm.at[:, pl.ds(0, PACK)],
                              gu_ar.at[pl.ds(gslot, K), :],
                              sem_sl.at[slot]).wait()
        pltpu.make_async_copy(tut_hbm.at[:, pl.ds(0, PACK)],
                              tu_ar.at[pl.ds(tslot, D), :],
                              sem_sl.at[slot]).wait()

    pltpu.sync_copy(pro_hbm.at[wid], pro_v)
    provec = pro_v[0, pl.ds(0, LANES)]
    for i in range(PF - 1):
        slab_fetch(provec[i], i)

    def chunk_body(c, rw_last):
        ch = wid * N_CHUNKS + c
        pltpu.sync_copy(meta_hbm.at[ch], meta_v)
        it_cp = pltpu.async_copy(it_hbm.at[meta_v.at[M_SI, 0]], it_v, sem_it)

        for g in range(GROUPS):
            sl = pl.ds(g * LANES, LANES)
            su_vec = meta_v[M_SU, 0, sl]
            nf_vec = meta_v[M_NF, 0, sl]
            rw_vec = meta_v[M_RW, 0, sl]
            pf_vec = meta_v[M_PF, 0, sl]
            for l in range(LANES):
                lg = g * LANES + l
                su_s = su_vec[l]
                w_slot = rw_vec[l]

                @pl.when(nf_vec[l] != 0)
                def _rotate():
                    slab_drain(w_slot)
                    slab_fetch(pf_vec[l], (w_slot + PF - 1) & (PF - 1))

                colv = jnp.full((LANES,), su_s & (PACK - 1), jnp.int32)
                gbase = w_slot * K
                for q in range(K // LANES):
                    guo_v[lg, pl.ds(q * LANES, LANES)] = plsc.load_gather(
                        gu_ar, [gbase + iot + q * LANES, colv])
                guo_v[lg, pl.ds(OUT_TU, D)] = plsc.load_gather(
                    tu_ar, [w_slot * D + iot, colv])

        it_cp.wait()

        for g in range(GROUPS):
            rows = iot + g * LANES
            acc = jnp.zeros((LANES,), jnp.float32)
            for k in range(K):
                acc = acc + (
                    plsc.load_gather(
                        guo_v, [rows, jnp.full((LANES,), k, jnp.int32)])
                    * plsc.load_gather(
                        it_v, [rows, jnp.full((LANES,), IT_GI + k, jnp.int32)]))
            for dd in range(D):
                acc = acc + (
                    plsc.load_gather(
                        guo_v, [rows, jnp.full((LANES,), OUT_TU + dd, jnp.int32)])
                    * plsc.load_gather(
                        it_v, [rows, jnp.full((LANES,), IT_FP + dd, jnp.int32)]))
            plsc.store_scatter(
                guo_v, [rows, jnp.full((LANES,), OUT_XUI, jnp.int32)], acc)

        pltpu.async_copy(guo_v, guo_hbm.at[meta_v.at[M_ORD, 0]],
                         sem_out).wait()
        pltpu.async_copy(it_v, ito_hbm.at[meta_v.at[M_ORD, 0]],
                         sem_out).wait()
        return meta_v[M_RW, 0, pl.ds(CH - LANES, LANES)][LANES - 1]

    rw_last = lax.fori_loop(0, N_CHUNKS, chunk_body, jnp.int32(0))

    for i in range(PF - 1):
        slab_drain((rw_last + 1 + i) & (PF - 1))


@functools.partial(
    pl.kernel,
    out_type=(
        jax.ShapeDtypeStruct((BATCH, PACK), jnp.float32),
        jax.ShapeDtypeStruct((BATCH, PACK), jnp.float32),
    ),
    mesh=plsc.VectorSubcoreMesh(core_axis_name="c", subcore_axis_name="s"),
    compiler_params=pltpu.CompilerParams(
        needs_layout_passes=False, use_tc_tiling_on_sc=True),
    scratch_types=[
        pltpu.VMEM((6, 1, CH), jnp.int32),
        pltpu.VMEM((1, PACK), jnp.int32),
        pltpu.VMEM((PF * K, PACK), jnp.float32),
        pltpu.VMEM((PF * D, PACK), jnp.float32),
        pltpu.VMEM((CH, PACK), jnp.float32),
        pltpu.VMEM((CH, PACK), jnp.float32),
        pltpu.SemaphoreType.DMA,
        pltpu.SemaphoreType.DMA,
        pltpu.SemaphoreType.DMA((PF,)),
    ],
)
def _sc_kernel(*refs):
    _sc_body(*refs)


def kernel(users, items, Gu, Gi, Tu, F, W, b):
    u = users[:, 0]
    it = items[:, 0]
    fp = _project(F, W, b)
    itab = jnp.pad(jnp.concatenate([F, fp, Gi], axis=1),
                   ((0, 0), (0, PACK - 2 * D - K)))

    pos = lax.iota(jnp.int32, BATCH)
    skey = jnp.sort(
        jnp.bitwise_or(jnp.left_shift(lax.shift_right_logical(u, 7), PBITS),
                       pos))
    order = jnp.bitwise_and(skey, (1 << PBITS) - 1)
    slab = lax.shift_right_logical(skey, PBITS)
    su = jnp.take(u, order)
    si = jnp.take(it, order)
    nf = jnp.where((pos % B_PER_W == 0) | (slab != jnp.roll(slab, 1)),
                   1, 0).astype(jnp.int32)
    runid = jnp.cumsum(nf) - 1
    sor = jnp.zeros((BATCH,), jnp.int32).at[runid].set(slab)
    pfs = jnp.take(sor, jnp.clip(runid + PF - 1, 0, BATCH - 1))
    runid0 = jnp.take(runid, (pos // B_PER_W) * B_PER_W)
    rw = (runid - runid0) & (PF - 1)
    meta = jnp.stack(
        [x.reshape(NCH, CH) for x in (su, si, order, nf, rw, pfs)],
        axis=1).reshape(NCH, 6, 1, CH)
    pro = jnp.take(sor, jnp.clip(
        runid[::B_PER_W][:, None]
        + jnp.arange(PF - 1, dtype=jnp.int32)[None, :], 0, BATCH - 1))
    pro3 = jnp.zeros((NW, 1, PACK), jnp.int32).at[:, 0, :PF - 1].set(pro)

    guo, ito = _sc_kernel(meta, pro3, Gu.T, Tu.T, itab)
    xui = guo[:, OUT_XUI]
    gamma_u = guo[:, :K]
    gamma_i = ito[:, IT_GI:IT_GI + K]
    theta_u = guo[:, OUT_TU:OUT_TU + D]
    effe_i = ito[:, IT_F:IT_F + D]
    return (xui, gamma_u, gamma_i, theta_u, effe_i)


# speedup vs baseline: 1.2564x; 1.1661x over previous
import functools

import jax
import jax.numpy as jnp
from jax import lax
from jax.experimental import pallas as pl
from jax.experimental.pallas import tpu as pltpu
from jax.experimental.pallas import tpu_sc as plsc

NUM_CORES = 2
NUM_SUBCORES = 16
LANES = 16
NW = NUM_CORES * NUM_SUBCORES

BATCH = 16384
K = 64
D = 16
PACK = 128
B_PER_W = BATCH // NW
CH = 64
N_CHUNKS = B_PER_W // CH
GROUPS = CH // LANES
NCH = BATCH // CH
PF = 8
PBITS = 18

M_SU = 0
M_SI = 1
M_ORD = 2
M_NF = 3
M_RW = 4
M_PF = 5

IT_F = 0
IT_FP = D
IT_GI = 2 * D
OUT_TU = K
OUT_XUI = K + D


def _project_body(f_ref, w_ref, b_ref, out_ref):
    out_ref[...] = lax.dot_general(
        f_ref[...], w_ref[...],
        dimension_numbers=(((1,), (1,)), ((), ())),
        preferred_element_type=jnp.float32,
    ) + b_ref[...]


def _project(F, W, b):
    return pl.pallas_call(
        _project_body,
        out_shape=jax.ShapeDtypeStruct((F.shape[0], W.shape[0]), jnp.float32),
    )(F, W, b.reshape(1, -1))


def _sc_body(meta_hbm, pro_hbm, gut_hbm, tut_hbm, it_hbm,
             guo_hbm, ito_hbm,
             meta_v, pro_v,
             gu_ar, tu_ar, it_v, guo_v,
             sem_it, sem_out, sem_sl):
    wid = lax.axis_index("s") * NUM_CORES + lax.axis_index("c")
    iot = lax.iota(jnp.int32, LANES)

    def slab_fetch(slab_id, slot):
        colb = pl.multiple_of(slab_id * PACK, PACK)
        gslot = pl.multiple_of(slot * K, K)
        tslot = pl.multiple_of(slot * D, D)
        pltpu.async_copy(gut_hbm.at[:, pl.ds(colb, PACK)],
                         gu_ar.at[pl.ds(gslot, K), :], sem_sl.at[slot])
        pltpu.async_copy(tut_hbm.at[:, pl.ds(colb, PACK)],
                         tu_ar.at[pl.ds(tslot, D), :], sem_sl.at[slot])

    def slab_drain(slot):
        gslot = pl.multiple_of(slot * K, K)
        tslot = pl.multiple_of(slot * D, D)
        pltpu.make_async_copy(gut_hbm.at[:, pl.ds(0, PACK)],
                              gu_ar.at[pl.ds(gslot, K), :],
                              sem_sl.at[slot]).wait()
        pltpu.make_async_copy(tut_hbm.at[:, pl.ds(0, PACK)],
                              tu_ar.at[pl.ds(tslot, D), :],
                              sem_sl.at[slot]).wait()

    pltpu.sync_copy(pro_hbm.at[wid], pro_v)
    provec = pro_v[0, pl.ds(0, LANES)]
    for i in range(PF - 1):
        slab_fetch(provec[i], i)

    def chunk_body(c, rw_last):
        ch = wid * N_CHUNKS + c
        pltpu.sync_copy(meta_hbm.at[ch], meta_v)
        it_cp = pltpu.async_copy(it_hbm.at[meta_v.at[M_SI, 0]], it_v, sem_it)

        for g in range(GROUPS):
            sl = pl.ds(g * LANES, LANES)
            su_vec = meta_v[M_SU, 0, sl]
            nf_vec = meta_v[M_NF, 0, sl]
            rw_vec = meta_v[M_RW, 0, sl]
            pf_vec = meta_v[M_PF, 0, sl]
            for l in range(LANES):
                lg = g * LANES + l
                su_s = su_vec[l]
                w_slot = rw_vec[l]

                @pl.when(nf_vec[l] != 0)
                def _rotate():
                    slab_drain(w_slot)
                    slab_fetch(pf_vec[l], (w_slot + PF - 1) & (PF - 1))

                colv = jnp.full((LANES,), su_s & (PACK - 1), jnp.int32)
                gbase = w_slot * K
                for q in range(K // LANES):
                    guo_v[lg, pl.ds(q * LANES, LANES)] = plsc.load_gather(
                        gu_ar, [gbase + iot + q * LANES, colv])
                guo_v[lg, pl.ds(OUT_TU, D)] = plsc.load_gather(
                    tu_ar, [w_slot * D + iot, colv])

        it_cp.wait()

        for g in range(GROUPS):
            rows = iot + g * LANES
            acc = jnp.zeros((LANES,), jnp.float32)
            for k in range(K):
                acc = acc + (
                    plsc.load_gather(
                        guo_v, [rows, jnp.full((LANES,), k, jnp.int32)])
                    * plsc.load_gather(
                        it_v, [rows, jnp.full((LANES,), IT_GI + k, jnp.int32)]))
            for dd in range(D):
                acc = acc + (
                    plsc.load_gather(
                        guo_v, [rows, jnp.full((LANES,), OUT_TU + dd, jnp.int32)])
                    * plsc.load_gather(
                        it_v, [rows, jnp.full((LANES,), IT_FP + dd, jnp.int32)]))
            plsc.store_scatter(
                guo_v, [rows, jnp.full((LANES,), OUT_XUI, jnp.int32)], acc)

        pltpu.async_copy(guo_v, guo_hbm.at[meta_v.at[M_ORD, 0]],
                         sem_out).wait()
        pltpu.async_copy(it_v, ito_hbm.at[meta_v.at[M_ORD, 0]],
                         sem_out).wait()
        return meta_v[M_RW, 0, pl.ds(CH - LANES, LANES)][LANES - 1]

    rw_last = lax.fori_loop(0, N_CHUNKS, chunk_body, jnp.int32(0))

    for i in range(PF - 1):
        slab_drain((rw_last + 1 + i) & (PF - 1))


@functools.partial(
    pl.kernel,
    out_type=(
        jax.ShapeDtypeStruct((BATCH, PACK), jnp.float32),
        jax.ShapeDtypeStruct((BATCH, PACK), jnp.float32),
    ),
    mesh=plsc.VectorSubcoreMesh(core_axis_name="c", subcore_axis_name="s"),
    compiler_params=pltpu.CompilerParams(
        needs_layout_passes=False, use_tc_tiling_on_sc=True),
    scratch_types=[
        pltpu.VMEM((6, 1, CH), jnp.int32),
        pltpu.VMEM((1, PACK), jnp.int32),
        pltpu.VMEM((PF * K, PACK), jnp.float32),
        pltpu.VMEM((PF * D, PACK), jnp.float32),
        pltpu.VMEM((CH, PACK), jnp.float32),
        pltpu.VMEM((CH, PACK), jnp.float32),
        pltpu.SemaphoreType.DMA,
        pltpu.SemaphoreType.DMA,
        pltpu.SemaphoreType.DMA((PF,)),
    ],
)
def _sc_kernel(*refs):
    _sc_body(*refs)


def kernel(users, items, Gu, Gi, Tu, F, W, b):
    u = users[:, 0]
    it = items[:, 0]
    fp = _project(F, W, b)
    itab = jnp.pad(jnp.concatenate([F, fp, Gi], axis=1),
                   ((0, 0), (0, PACK - 2 * D - K)))

    pos = lax.iota(jnp.int32, BATCH)
    skey = jnp.sort(
        jnp.bitwise_or(jnp.left_shift(lax.shift_right_logical(u, 7), PBITS),
                       pos))
    order = jnp.bitwise_and(skey, (1 << PBITS) - 1)
    slab = lax.shift_right_logical(skey, PBITS)
    su = jnp.take(u, order)
    si = jnp.take(it, order)
    nf = jnp.where((pos % B_PER_W == 0) | (slab != jnp.roll(slab, 1)),
                   1, 0).astype(jnp.int32)
    runid = jnp.cumsum(nf) - 1
    nsp = jnp.sort(jnp.where(nf == 1, pos, BATCH - 1))
    pfs = jnp.take(slab, jnp.take(
        nsp, jnp.clip(runid + PF - 1, 0, BATCH - 1)))
    runid0 = jnp.take(runid, (pos // B_PER_W) * B_PER_W)
    rw = (runid - runid0) & (PF - 1)
    meta = jnp.stack(
        [x.reshape(NCH, CH) for x in (su, si, order, nf, rw, pfs)],
        axis=1).reshape(NCH, 6, 1, CH)
    pro = jnp.take(slab, jnp.take(nsp, jnp.clip(
        runid[::B_PER_W][:, None]
        + jnp.arange(PF - 1, dtype=jnp.int32)[None, :], 0, BATCH - 1)))
    pro3 = jnp.zeros((NW, 1, PACK), jnp.int32).at[:, 0, :PF - 1].set(pro)

    guo, ito = _sc_kernel(meta, pro3, Gu.T, Tu.T, itab)
    xui = guo[:, OUT_XUI]
    gamma_u = guo[:, :K]
    gamma_i = ito[:, IT_GI:IT_GI + K]
    theta_u = guo[:, OUT_TU:OUT_TU + D]
    effe_i = ito[:, IT_F:IT_F + D]
    return (xui, gamma_u, gamma_i, theta_u, effe_i)
